# Initial kernel scaffold; baseline (speedup 1.0000x reference)
import jax
import jax.numpy as jnp
from jax.experimental import pallas as pl


def kernel(x, embed_table):
    raise NotImplementedError("write your pallas kernel here")


# speedup vs baseline: 1.5362x; 1.5362x over previous
import functools

import jax
import jax.numpy as jnp
from jax import lax
from jax.experimental import pallas as pl
from jax.experimental.pallas import tpu as pltpu
from jax.experimental.pallas import tpu_sc as plsc

BATCH = 32
SEQ = 512
HIDDEN = 896
NTOK = BATCH * SEQ
NC = 2
NS = 16
NW = NC * NS
TOK_PER_W = NTOK // NW
CHUNK = 64
NCHUNK = TOK_PER_W // CHUNK

_mesh = plsc.VectorSubcoreMesh(core_axis_name="c", subcore_axis_name="s")


@functools.partial(
    pl.kernel,
    mesh=_mesh,
    out_type=jax.ShapeDtypeStruct((NTOK, HIDDEN), jnp.float32),
    scratch_types=[
        pltpu.VMEM((NCHUNK, CHUNK), jnp.int32),
        pltpu.VMEM((2, CHUNK, HIDDEN), jnp.float32),
        pltpu.SemaphoreType.DMA,
        pltpu.SemaphoreType.DMA,
        pltpu.SemaphoreType.DMA,
        pltpu.SemaphoreType.DMA,
    ],
)
def _embed_lookup(idx_hbm, tab_hbm, out_hbm, idx_v, rows_v, g0, g1, w0, w1):
    wid = lax.axis_index("s") * NC + lax.axis_index("c")
    base = wid * TOK_PER_W
    pltpu.sync_copy(idx_hbm.at[wid], idx_v)

    gsems = (g0, g1)
    wsems = (w0, w1)
    writes = [None, None]

    gather = pltpu.async_copy(tab_hbm.at[idx_v.at[0]], rows_v.at[0], gsems[0])
    for c in range(NCHUNK):
        s = c % 2
        ns = (c + 1) % 2
        gather.wait()
        writes[s] = pltpu.async_copy(
            rows_v.at[s], out_hbm.at[pl.ds(base + c * CHUNK, CHUNK)], wsems[s]
        )
        if c + 1 < NCHUNK:
            if writes[ns] is not None:
                writes[ns].wait()
            gather = pltpu.async_copy(
                tab_hbm.at[idx_v.at[c + 1]], rows_v.at[ns], gsems[ns]
            )
    writes[0].wait()
    writes[1].wait()


def kernel(x, embed_table):
    idx = x.reshape(NW, NCHUNK, CHUNK).astype(jnp.int32)
    out = _embed_lookup(idx, embed_table)
    return out.reshape(BATCH, SEQ, HIDDEN)


# speedup vs baseline: 1.5431x; 1.0045x over previous
import functools

import jax
import jax.numpy as jnp
from jax import lax
from jax.experimental import pallas as pl
from jax.experimental.pallas import tpu as pltpu
from jax.experimental.pallas import tpu_sc as plsc

BATCH = 32
SEQ = 512
HIDDEN = 896
NTOK = BATCH * SEQ
NC = 2
NS = 16
NW = NC * NS
TOK_PER_W = NTOK // NW
CHUNK = 64
NCHUNK = TOK_PER_W // CHUNK

_mesh = plsc.VectorSubcoreMesh(core_axis_name="c", subcore_axis_name="s")


@functools.partial(
    pl.kernel,
    mesh=_mesh,
    out_type=jax.ShapeDtypeStruct((BATCH, SEQ, HIDDEN), jnp.float32),
    scratch_types=[
        pltpu.VMEM((TOK_PER_W,), jnp.int32),
        pltpu.VMEM((2, CHUNK, HIDDEN), jnp.float32),
        pltpu.SemaphoreType.DMA,
        pltpu.SemaphoreType.DMA,
        pltpu.SemaphoreType.DMA,
        pltpu.SemaphoreType.DMA,
    ],
)
def _embed_lookup(idx_hbm, tab_hbm, out_hbm, idx_v, rows_v, g0, g1, w0, w1):
    wid = lax.axis_index("s") * NC + lax.axis_index("c")
    pltpu.sync_copy(idx_hbm.at[wid], idx_v)

    gsems = (g0, g1)
    wsems = (w0, w1)
    writes = [None, None]

    gather = pltpu.async_copy(
        tab_hbm.at[idx_v.at[pl.ds(0, CHUNK)]], rows_v.at[0], gsems[0]
    )
    for c in range(NCHUNK):
        s = c % 2
        ns = (c + 1) % 2
        gather.wait()
        writes[s] = pltpu.async_copy(
            rows_v.at[s], out_hbm.at[wid, pl.ds(c * CHUNK, CHUNK)], wsems[s]
        )
        if c + 1 < NCHUNK:
            if writes[ns] is not None:
                writes[ns].wait()
            gather = pltpu.async_copy(
                tab_hbm.at[idx_v.at[pl.ds((c + 1) * CHUNK, CHUNK)]],
                rows_v.at[ns],
                gsems[ns],
            )
    writes[0].wait()
    writes[1].wait()


def kernel(x, embed_table):
    return _embed_lookup(x.astype(jnp.int32), embed_table)


# speedup vs baseline: 1.6194x; 1.0494x over previous
import functools

import jax
import jax.numpy as jnp
from jax import lax
from jax.experimental import pallas as pl
from jax.experimental.pallas import tpu as pltpu
from jax.experimental.pallas import tpu_sc as plsc

BATCH = 32
SEQ = 512
HIDDEN = 896
NTOK = BATCH * SEQ
NC = 2
NS = 16
NW = NC * NS
TOK_PER_W = NTOK // NW
CHUNK = 32
NCHUNK = TOK_PER_W // CHUNK
NBUF = 4

_mesh = plsc.VectorSubcoreMesh(core_axis_name="c", subcore_axis_name="s")


@functools.partial(
    pl.kernel,
    mesh=_mesh,
    out_type=jax.ShapeDtypeStruct((BATCH, SEQ, HIDDEN), jnp.float32),
    scratch_types=[
        pltpu.VMEM((TOK_PER_W,), jnp.int32),
        pltpu.VMEM((NBUF, CHUNK, HIDDEN), jnp.float32),
    ]
    + [pltpu.SemaphoreType.DMA] * (2 * NBUF),
)
def _embed_lookup(idx_hbm, tab_hbm, out_hbm, idx_v, rows_v, *sems):
    wid = lax.axis_index("s") * NC + lax.axis_index("c")
    pltpu.sync_copy(idx_hbm.at[wid], idx_v)

    gsems = sems[:NBUF]
    wsems = sems[NBUF:]
    gathers = [None] * NBUF
    writes = [None] * NBUF

    def start_gather(c):
        s = c % NBUF
        gathers[s] = pltpu.async_copy(
            tab_hbm.at[idx_v.at[pl.ds(c * CHUNK, CHUNK)]], rows_v.at[s], gsems[s]
        )

    for c in range(min(NBUF, NCHUNK)):
        start_gather(c)
    for c in range(NCHUNK):
        s = c % NBUF
        gathers[s].wait()
        writes[s] = pltpu.async_copy(
            rows_v.at[s], out_hbm.at[wid, pl.ds(c * CHUNK, CHUNK)], wsems[s]
        )
        if c + NBUF < NCHUNK:
            writes[s].wait()
            start_gather(c + NBUF)
    for w in writes:
        if w is not None:
            w.wait()


def kernel(x, embed_table):
    return _embed_lookup(x.astype(jnp.int32), embed_table)
